# baseline (speedup 1.0000x reference)
import functools
import math

import jax
import jax.numpy as jnp
from jax import lax
from jax.experimental import pallas as pl
from jax.experimental.pallas import tpu as pltpu
from jax.experimental.pallas import tpu_sc as plsc

VOCAB = 100000
HID = 64
PAD = -100


_LN_ROWS = 2000


def _ln_body(t_ref, g_ref, b_ref, o_ref):
    h = t_ref[:] * math.sqrt(float(HID))
    m = jnp.mean(h, axis=1, keepdims=True)
    d = h - m
    v = jnp.mean(d * d, axis=1, keepdims=True)
    y = d * lax.rsqrt(v + 1e-5) * g_ref[:] + b_ref[:]
    y3 = y.reshape(_LN_ROWS // 2, 2, HID)
    o_ref[:] = jnp.concatenate([y3[:, 0, :], y3[:, 1, :]], axis=1)


def _normalize_table(table, gamma, beta):
    g2 = gamma.reshape(1, HID)
    b2 = beta.reshape(1, HID)
    return pl.pallas_call(
        _ln_body,
        grid=(VOCAB // _LN_ROWS,),
        in_specs=[
            pl.BlockSpec((_LN_ROWS, HID), lambda i: (i, 0)),
            pl.BlockSpec((1, HID), lambda i: (0, 0)),
            pl.BlockSpec((1, HID), lambda i: (0, 0)),
        ],
        out_specs=pl.BlockSpec((_LN_ROWS // 2, 2 * HID), lambda i: (i, 0)),
        out_shape=jax.ShapeDtypeStruct((VOCAB // 2, 2 * HID), jnp.float32),
    )(table, g2, b2)


_NC = 2
_NS = 16
_NW = _NC * _NS
_NSEQ = 16384
_SEQ = 50
_SL_W = _NSEQ // _NW
_NCHUNK = _SL_W // 2
_G = 2 * _SEQ
_L = 16


@functools.partial(
    pl.kernel,
    mesh=plsc.VectorSubcoreMesh(core_axis_name="c", subcore_axis_name="s"),
    out_type=jax.ShapeDtypeStruct((_NSEQ, _SEQ, HID), jnp.float32),
    scratch_types=[
        pltpu.VMEM((_NCHUNK, _G), jnp.int32),
        pltpu.VMEM((_NCHUNK, _G), jnp.int32),
        [pltpu.VMEM((_G, 2 * HID), jnp.float32) for _ in range(2)],
        [pltpu.VMEM((_SEQ, HID), jnp.float32) for _ in range(4)],
        [pltpu.SemaphoreType.DMA for _ in range(2)],
        [pltpu.SemaphoreType.DMA for _ in range(2)],
    ],
    compiler_params=pltpu.CompilerParams(use_tc_tiling_on_sc=True, needs_layout_passes=False),
)
def _gather_k(idx_hbm, tab_hbm, out_hbm, idx_v, half_v, abufs, bbufs, gsems, wsems):
    wid = lax.axis_index("s") * _NC + lax.axis_index("c")
    sl0 = wid * _SL_W
    pltpu.sync_copy(idx_hbm.at[wid], idx_v)

    def derive(j, carry):
        for seg in range(_G // _L):
            vv = idx_v[j, pl.ds(seg * _L, _L)]
            half_v[j, pl.ds(seg * _L, _L)] = jnp.right_shift(vv, 1)
        vv = idx_v[j, pl.ds(_G - _L, _L)]
        half_v[j, pl.ds(_G - _L, _L)] = jnp.right_shift(vv, 1)
        return carry

    lax.fori_loop(0, _NCHUNK, derive, 0)

    def fire(ch, slot):
        pltpu.async_copy(tab_hbm.at[half_v.at[ch]], abufs[slot], gsems[slot])

    def drain_gather(slot):
        pltpu.make_async_copy(
            tab_hbm.at[half_v.at[0]], abufs[slot], gsems[slot]
        ).wait()

    def repack(ch, slot, s, bb):
        a = abufs[slot]
        lanes = lax.iota(jnp.int32, _L)

        def group(r0, num):
            rows = r0 + lanes
            mask = lanes < num
            vv = plsc.load_gather(idx_v, [jnp.full((_L,), ch, jnp.int32),
                                          s * _SEQ + rows], mask=mask)
            off = jnp.left_shift(jnp.bitwise_and(vv, 1), 6)
            for c in range(HID):
                val = plsc.load_gather(a, [s * _SEQ + rows, off + c], mask=mask)
                plsc.store_scatter(bb, [rows, jnp.full((_L,), c, jnp.int32)],
                                   val, mask=mask)

        def rows_loop(g, carry):
            group(g * _L, _L)
            return carry

        lax.fori_loop(0, _SEQ // _L, rows_loop, 0)
        group((_SEQ // _L) * _L, _SEQ % _L)

    def write(i, bb, slot):
        pltpu.async_copy(bb, out_hbm.at[i], wsems[slot])

    def drain_write(bb, slot):
        pltpu.make_async_copy(bb, out_hbm.at[0], wsems[slot]).wait()

    fire(0, 0)
    fire(1, 1)

    def body(p, carry):
        for slot in range(2):
            ch = 2 * p + slot
            drain_gather(slot)

            @pl.when(p > 0)
            def _():
                drain_write(bbufs[2 * slot], slot)
                drain_write(bbufs[2 * slot + 1], slot)

            repack(ch, slot, 0, bbufs[2 * slot])
            repack(ch, slot, 1, bbufs[2 * slot + 1])
            write(sl0 + 2 * ch, bbufs[2 * slot], slot)
            write(sl0 + 2 * ch + 1, bbufs[2 * slot + 1], slot)

            @pl.when(p < _NCHUNK // 2 - 1)
            def _():
                fire(ch + 2, slot)

        return carry

    lax.fori_loop(0, _NCHUNK // 2, body, 0)

    for slot in range(2):
        drain_write(bbufs[2 * slot], slot)
        drain_write(bbufs[2 * slot + 1], slot)


def kernel(x, table, gamma, beta):
    table_p = _normalize_table(table, gamma, beta)
    x_mapped = jnp.where(x == PAD, VOCAB - 100, x)
    x_mapped = jnp.clip(x_mapped, 0, VOCAB - 1)
    idx3 = x_mapped.reshape(_NW, _NCHUNK, _G)
    return _gather_k(idx3, table_p)


# speedup vs baseline: 3.3057x; 3.3057x over previous
import functools
import math

import jax
import jax.numpy as jnp
from jax import lax
from jax.experimental import pallas as pl
from jax.experimental.pallas import tpu as pltpu
from jax.experimental.pallas import tpu_sc as plsc

VOCAB = 100000
HID = 64
PAD = -100


_LN_ROWS = 2000


def _ln_body(t_ref, g_ref, b_ref, o_ref):
    h = t_ref[:] * math.sqrt(float(HID))
    m = jnp.mean(h, axis=1, keepdims=True)
    d = h - m
    v = jnp.mean(d * d, axis=1, keepdims=True)
    y = d * lax.rsqrt(v + 1e-5) * g_ref[:] + b_ref[:]
    o_ref[:] = jnp.concatenate([y, jnp.zeros_like(y)], axis=1)


def _normalize_table(table, gamma, beta):
    g2 = gamma.reshape(1, HID)
    b2 = beta.reshape(1, HID)
    return pl.pallas_call(
        _ln_body,
        grid=(VOCAB // _LN_ROWS,),
        in_specs=[
            pl.BlockSpec((_LN_ROWS, HID), lambda i: (i, 0)),
            pl.BlockSpec((1, HID), lambda i: (0, 0)),
            pl.BlockSpec((1, HID), lambda i: (0, 0)),
        ],
        out_specs=pl.BlockSpec((_LN_ROWS, 2 * HID), lambda i: (i, 0)),
        out_shape=jax.ShapeDtypeStruct((VOCAB, 2 * HID), jnp.float32),
    )(table, g2, b2)


_NC = 2
_NS = 16
_NW = _NC * _NS
_NSEQ = 16384
_SEQ = 50
_SL_W = _NSEQ // _NW
_NCHUNK = _SL_W // 2
_G = 2 * _SEQ


@functools.partial(
    pl.kernel,
    mesh=plsc.VectorSubcoreMesh(core_axis_name="c", subcore_axis_name="s"),
    out_type=jax.ShapeDtypeStruct((_NSEQ, _SEQ, HID), jnp.float32),
    scratch_types=[
        pltpu.VMEM((_NCHUNK, _G), jnp.int32),
        [pltpu.VMEM((_G, 2 * HID), jnp.float32) for _ in range(2)],
        [pltpu.VMEM((_SEQ, HID), jnp.float32) for _ in range(4)],
        [pltpu.SemaphoreType.DMA for _ in range(2)],
        [pltpu.SemaphoreType.DMA for _ in range(2)],
    ],
    compiler_params=pltpu.CompilerParams(use_tc_tiling_on_sc=True),
)
def _gather_k(idx_hbm, tab_hbm, out_hbm, idx_v, abufs, bbufs, gsems, wsems):
    wid = lax.axis_index("s") * _NC + lax.axis_index("c")
    sl0 = wid * _SL_W
    pltpu.sync_copy(idx_hbm.at[wid], idx_v)

    def fire(ch, slot):
        pltpu.async_copy(tab_hbm.at[idx_v.at[ch]], abufs[slot], gsems[slot])

    def drain_gather(slot):
        pltpu.make_async_copy(
            tab_hbm.at[idx_v.at[0]], abufs[slot], gsems[slot]
        ).wait()

    def repack(slot, s, bb):
        a = abufs[slot]

        def rows(rg, carry):
            for rr in range(10):
                r = rg * 10 + rr
                for k in range(HID // 16):
                    bb[r, pl.ds(k * 16, 16)] = a[s * _SEQ + r, pl.ds(k * 16, 16)]
            return carry

        lax.fori_loop(0, _SEQ // 10, rows, 0)

    def write(i, bb, slot):
        pltpu.async_copy(bb, out_hbm.at[i], wsems[slot])

    def drain_write(bb, slot):
        pltpu.make_async_copy(bb, out_hbm.at[0], wsems[slot]).wait()

    fire(0, 0)
    fire(1, 1)

    def body(p, carry):
        for slot in range(2):
            ch = 2 * p + slot
            drain_gather(slot)

            @pl.when(p > 0)
            def _():
                drain_write(bbufs[2 * slot], slot)
                drain_write(bbufs[2 * slot + 1], slot)

            repack(slot, 0, bbufs[2 * slot])
            repack(slot, 1, bbufs[2 * slot + 1])
            write(sl0 + 2 * ch, bbufs[2 * slot], slot)
            write(sl0 + 2 * ch + 1, bbufs[2 * slot + 1], slot)

            @pl.when(p < _NCHUNK // 2 - 1)
            def _():
                fire(ch + 2, slot)

        return carry

    lax.fori_loop(0, _NCHUNK // 2, body, 0)

    for slot in range(2):
        drain_write(bbufs[2 * slot], slot)
        drain_write(bbufs[2 * slot + 1], slot)


def kernel(x, table, gamma, beta):
    table_p = _normalize_table(table, gamma, beta)
    x_mapped = jnp.where(x == PAD, VOCAB - 100, x)
    x_mapped = jnp.clip(x_mapped, 0, VOCAB - 1)
    idx3 = x_mapped.reshape(_NW, _NCHUNK, _G)
    return _gather_k(idx3, table_p)
